# baseline (device time: 128242 ns/iter reference)
import jax
import jax.numpy as jnp
from jax import lax
from jax.experimental import pallas as pl
from jax.experimental.pallas import tpu as pltpu

N_DEV = 4


def kernel(A, B):
    m_per, k = A.shape
    _, n = B.shape
    half = m_per // 2

    a_bf = A.astype(jnp.bfloat16)
    b_bf = B.astype(jnp.bfloat16)

    def body(abf_ref, bbf_ref, out_ref,
             cl_ref, cr_ref, co_ref, send_sems, recv_sems):
        my = lax.axis_index("i")
        left = lax.rem(my + N_DEV - 1, N_DEV)
        right = lax.rem(my + 1, N_DEV)
        opp = lax.rem(my + 2, N_DEV)

        barrier_sem = pltpu.get_barrier_semaphore()
        for nbr in (left, right):
            pl.semaphore_signal(
                barrier_sem, inc=1,
                device_id=(nbr,), device_id_type=pl.DeviceIdType.MESH,
            )
        pl.semaphore_wait(barrier_sem, 2)

        def rdma(src, dst, sem_idx, target):
            return pltpu.make_async_remote_copy(
                src_ref=src, dst_ref=dst,
                send_sem=send_sems.at[sem_idx], recv_sem=recv_sems.at[sem_idx],
                device_id=(target,), device_id_type=pl.DeviceIdType.MESH,
            )

        lo = pl.ds(0, half)
        hi = pl.ds(half, half)

        r0 = rdma(abf_ref.at[lo], cl_ref.at[lo], 0, right)
        r1 = rdma(cl_ref.at[lo], co_ref.at[lo], 1, right)
        r2 = rdma(abf_ref.at[hi], cl_ref.at[hi], 2, right)
        l0 = rdma(abf_ref.at[hi], cr_ref.at[hi], 3, left)
        l1 = rdma(cr_ref.at[hi], co_ref.at[hi], 4, left)
        l2 = rdma(abf_ref.at[lo], cr_ref.at[lo], 5, left)

        r0.start()
        l0.start()

        def mm(chunk, row0, nrows):
            out_ref[pl.ds(row0, nrows), :] = jnp.dot(
                chunk, bbf_ref[...], preferred_element_type=jnp.float32
            ).astype(jnp.bfloat16)

        mm(abf_ref[...], my * m_per, m_per)

        r0.wait_recv()
        r1.start()
        r2.start()
        mm(cl_ref[lo, :], left * m_per, half)

        l0.wait_recv()
        l1.start()
        l2.start()
        mm(cr_ref[hi, :], right * m_per + half, half)

        r1.wait_recv()
        mm(co_ref[lo, :], opp * m_per, half)

        l1.wait_recv()
        mm(co_ref[hi, :], opp * m_per + half, half)

        r2.wait_recv()
        mm(cl_ref[hi, :], left * m_per + half, half)

        l2.wait_recv()
        mm(cr_ref[lo, :], right * m_per, half)

        for t in (r0, r1, r2, l0, l1, l2):
            t.wait_send()

    return pl.pallas_call(
        body,
        out_shape=jax.ShapeDtypeStruct((N_DEV * m_per, n), jnp.bfloat16),
        in_specs=[
            pl.BlockSpec(memory_space=pltpu.VMEM),
            pl.BlockSpec(memory_space=pltpu.VMEM),
        ],
        out_specs=pl.BlockSpec(memory_space=pltpu.VMEM),
        scratch_shapes=[
            pltpu.VMEM((m_per, k), jnp.bfloat16),
            pltpu.VMEM((m_per, k), jnp.bfloat16),
            pltpu.VMEM((m_per, k), jnp.bfloat16),
            pltpu.SemaphoreType.DMA((6,)),
            pltpu.SemaphoreType.DMA((6,)),
        ],
        compiler_params=pltpu.CompilerParams(
            collective_id=0,
            vmem_limit_bytes=100 * 1024 * 1024,
        ),
    )(a_bf, b_bf)


# device time: 118354 ns/iter; 1.0835x vs baseline; 1.0835x over previous
import jax
import jax.numpy as jnp
from jax import lax
from jax.experimental import pallas as pl
from jax.experimental.pallas import tpu as pltpu

N_DEV = 4


def kernel(A, B):
    m_per, k = A.shape
    _, n = B.shape
    half = m_per // 2

    a_bf = A.astype(jnp.bfloat16)
    b_bf = B.astype(jnp.bfloat16)
    def body(a_ref, b_ref, out_ref, cl_ref, cr_ref, co_ref,
             stage_ref, send_sems, recv_sems, store_sems):
        my = lax.axis_index("i")
        left = lax.rem(my + N_DEV - 1, N_DEV)
        right = lax.rem(my + 1, N_DEV)
        opp = lax.rem(my + 2, N_DEV)

        barrier_sem = pltpu.get_barrier_semaphore()
        for nbr in (left, right):
            pl.semaphore_signal(
                barrier_sem, inc=1,
                device_id=(nbr,), device_id_type=pl.DeviceIdType.MESH,
            )
        pl.semaphore_wait(barrier_sem, 2)

        def rdma(src, dst, sem_idx, target):
            return pltpu.make_async_remote_copy(
                src_ref=src, dst_ref=dst,
                send_sem=send_sems.at[sem_idx], recv_sem=recv_sems.at[sem_idx],
                device_id=(target,), device_id_type=pl.DeviceIdType.MESH,
            )

        lo = pl.ds(0, half)
        hi = pl.ds(half, half)
        t1r_lo = rdma(a_ref.at[lo], cl_ref.at[lo], 0, right)
        t1r_hi = rdma(a_ref.at[hi], cl_ref.at[hi], 1, right)
        t1l_hi = rdma(a_ref.at[hi], cr_ref.at[hi], 2, left)
        t1l_lo = rdma(a_ref.at[lo], cr_ref.at[lo], 3, left)
        t2r = rdma(cl_ref.at[lo], co_ref.at[lo], 4, right)
        t2l = rdma(cr_ref.at[hi], co_ref.at[hi], 5, left)

        t1r_lo.start()
        t1l_hi.start()
        t1r_hi.start()
        t1l_lo.start()

        stores = [None] * 8

        def mm(idx, chunk, row0):
            slot = idx % 2
            if stores[idx - 2] is not None:
                stores[idx - 2].wait()
            stage_ref[slot] = jnp.dot(
                chunk, b_ref[...], preferred_element_type=jnp.float32
            ).astype(jnp.bfloat16)
            cp = pltpu.make_async_copy(
                stage_ref.at[slot],
                out_ref.at[pl.ds(row0, half), :],
                store_sems.at[idx],
            )
            cp.start()
            stores[idx] = cp

        mm(0, a_ref[lo, :], my * m_per)
        mm(1, a_ref[hi, :], my * m_per + half)

        t1r_lo.wait_recv()
        t2r.start()
        mm(2, cl_ref[lo, :], left * m_per)

        t1l_hi.wait_recv()
        t2l.start()
        mm(3, cr_ref[hi, :], right * m_per + half)

        t1r_hi.wait_recv()
        mm(4, cl_ref[hi, :], left * m_per + half)

        t1l_lo.wait_recv()
        mm(5, cr_ref[lo, :], right * m_per)

        t2r.wait_recv()
        mm(6, co_ref[lo, :], opp * m_per)

        t2l.wait_recv()
        mm(7, co_ref[hi, :], opp * m_per + half)

        stores[6].wait()
        stores[7].wait()

        for t in (t1r_lo, t1r_hi, t1l_hi, t1l_lo, t2r, t2l):
            t.wait_send()

    return pl.pallas_call(
        body,
        out_shape=jax.ShapeDtypeStruct((N_DEV * m_per, n), jnp.bfloat16),
        in_specs=[
            pl.BlockSpec(memory_space=pltpu.VMEM),
            pl.BlockSpec(memory_space=pltpu.VMEM),
        ],
        out_specs=pl.BlockSpec(memory_space=pl.MemorySpace.ANY),
        scratch_shapes=[
            pltpu.VMEM((m_per, k), jnp.bfloat16),
            pltpu.VMEM((m_per, k), jnp.bfloat16),
            pltpu.VMEM((m_per, k), jnp.bfloat16),
            pltpu.VMEM((2, half, n), jnp.bfloat16),
            pltpu.SemaphoreType.DMA((6,)),
            pltpu.SemaphoreType.DMA((6,)),
            pltpu.SemaphoreType.DMA((8,)),
        ],
        compiler_params=pltpu.CompilerParams(
            collective_id=0,
            vmem_limit_bytes=100 * 1024 * 1024,
        ),
    )(a_bf, b_bf)


# device time: 115307 ns/iter; 1.1122x vs baseline; 1.0264x over previous
import jax
import jax.numpy as jnp
from jax import lax
from jax.experimental import pallas as pl
from jax.experimental.pallas import tpu as pltpu

N_DEV = 4


def kernel(A, B):
    m_per, k = A.shape
    _, n = B.shape
    half = m_per // 2

    a_bf = A.astype(jnp.bfloat16)

    def body(a_ref, b32_ref, out_ref, cl_ref, cr_ref, co_ref,
             stage_ref, bbf_ref, send_sems, recv_sems, store_sems):
        my = lax.axis_index("i")
        left = lax.rem(my + N_DEV - 1, N_DEV)
        right = lax.rem(my + 1, N_DEV)
        opp = lax.rem(my + 2, N_DEV)

        barrier_sem = pltpu.get_barrier_semaphore()
        for nbr in (left, right):
            pl.semaphore_signal(
                barrier_sem, inc=1,
                device_id=(nbr,), device_id_type=pl.DeviceIdType.MESH,
            )
        pl.semaphore_wait(barrier_sem, 2)

        def rdma(src, dst, sem_idx, target):
            return pltpu.make_async_remote_copy(
                src_ref=src, dst_ref=dst,
                send_sem=send_sems.at[sem_idx], recv_sem=recv_sems.at[sem_idx],
                device_id=(target,), device_id_type=pl.DeviceIdType.MESH,
            )

        lo = pl.ds(0, half)
        hi = pl.ds(half, half)
        t1r_lo = rdma(a_ref.at[lo], cl_ref.at[lo], 0, right)
        t1r_hi = rdma(a_ref.at[hi], cl_ref.at[hi], 1, right)
        t1l_hi = rdma(a_ref.at[hi], cr_ref.at[hi], 2, left)
        t1l_lo = rdma(a_ref.at[lo], cr_ref.at[lo], 3, left)
        t2r = rdma(cl_ref.at[lo], co_ref.at[lo], 4, right)
        t2l = rdma(cr_ref.at[hi], co_ref.at[hi], 5, left)

        t1r_lo.start()
        t1l_hi.start()
        t1r_hi.start()
        t1l_lo.start()

        bbf_ref[...] = b32_ref[...].astype(jnp.bfloat16)

        stores = [None] * 8

        def mm(idx, chunk, row0):
            slot = idx % 2
            if stores[idx - 2] is not None:
                stores[idx - 2].wait()
            stage_ref[slot] = jnp.dot(
                chunk, bbf_ref[...], preferred_element_type=jnp.float32
            ).astype(jnp.bfloat16)
            cp = pltpu.make_async_copy(
                stage_ref.at[slot],
                out_ref.at[pl.ds(row0, half), :],
                store_sems.at[idx],
            )
            cp.start()
            stores[idx] = cp

        mm(0, a_ref[lo, :], my * m_per)
        mm(1, a_ref[hi, :], my * m_per + half)

        t1r_lo.wait_recv()
        t2r.start()
        mm(2, cl_ref[lo, :], left * m_per)

        t1l_hi.wait_recv()
        t2l.start()
        mm(3, cr_ref[hi, :], right * m_per + half)

        t1r_hi.wait_recv()
        mm(4, cl_ref[hi, :], left * m_per + half)

        t1l_lo.wait_recv()
        mm(5, cr_ref[lo, :], right * m_per)

        t2r.wait_recv()
        mm(6, co_ref[lo, :], opp * m_per)

        t2l.wait_recv()
        mm(7, co_ref[hi, :], opp * m_per + half)

        stores[6].wait()
        stores[7].wait()

        for t in (t1r_lo, t1r_hi, t1l_hi, t1l_lo, t2r, t2l):
            t.wait_send()

    return pl.pallas_call(
        body,
        out_shape=jax.ShapeDtypeStruct((N_DEV * m_per, n), jnp.bfloat16),
        in_specs=[
            pl.BlockSpec(memory_space=pltpu.VMEM),
            pl.BlockSpec(memory_space=pltpu.VMEM),
        ],
        out_specs=pl.BlockSpec(memory_space=pl.MemorySpace.ANY),
        scratch_shapes=[
            pltpu.VMEM((m_per, k), jnp.bfloat16),
            pltpu.VMEM((m_per, k), jnp.bfloat16),
            pltpu.VMEM((m_per, k), jnp.bfloat16),
            pltpu.VMEM((2, half, n), jnp.bfloat16),
            pltpu.VMEM((k, n), jnp.bfloat16),
            pltpu.SemaphoreType.DMA((6,)),
            pltpu.SemaphoreType.DMA((6,)),
            pltpu.SemaphoreType.DMA((8,)),
        ],
        compiler_params=pltpu.CompilerParams(
            collective_id=0,
            vmem_limit_bytes=100 * 1024 * 1024,
        ),
    )(a_bf, B)


# device time: 110464 ns/iter; 1.1609x vs baseline; 1.0438x over previous
import jax
import jax.numpy as jnp
from jax import lax
from jax.experimental import pallas as pl
from jax.experimental.pallas import tpu as pltpu

N_DEV = 4


def kernel(A, B):
    m_per, k = A.shape
    _, n = B.shape
    half = m_per // 2

    def body(a32_ref, b32_ref, out_ref, cl_ref, cr_ref, co_ref,
             stage_ref, bbf_ref, a_ref, send_sems, recv_sems, store_sems):
        my = lax.axis_index("i")
        left = lax.rem(my + N_DEV - 1, N_DEV)
        right = lax.rem(my + 1, N_DEV)
        opp = lax.rem(my + 2, N_DEV)

        barrier_sem = pltpu.get_barrier_semaphore()
        for nbr in (left, right):
            pl.semaphore_signal(
                barrier_sem, inc=1,
                device_id=(nbr,), device_id_type=pl.DeviceIdType.MESH,
            )
        pl.semaphore_wait(barrier_sem, 2)

        def rdma(src, dst, sem_idx, target):
            return pltpu.make_async_remote_copy(
                src_ref=src, dst_ref=dst,
                send_sem=send_sems.at[sem_idx], recv_sem=recv_sems.at[sem_idx],
                device_id=(target,), device_id_type=pl.DeviceIdType.MESH,
            )

        lo = pl.ds(0, half)
        hi = pl.ds(half, half)
        t1r_lo = rdma(a_ref.at[lo], cl_ref.at[lo], 0, right)
        t1r_hi = rdma(a_ref.at[hi], cl_ref.at[hi], 1, right)
        t1l_hi = rdma(a_ref.at[hi], cr_ref.at[hi], 2, left)
        t1l_lo = rdma(a_ref.at[lo], cr_ref.at[lo], 3, left)
        t2r = rdma(cl_ref.at[lo], co_ref.at[lo], 4, right)
        t2l = rdma(cr_ref.at[hi], co_ref.at[hi], 5, left)

        a_ref[...] = a32_ref[...].astype(jnp.bfloat16)

        t1r_lo.start()
        t1l_hi.start()
        t1r_hi.start()
        t1l_lo.start()

        bbf_ref[...] = b32_ref[...].astype(jnp.bfloat16)

        stores = [None] * 8

        def mm(idx, chunk, row0):
            slot = idx % 2
            if stores[idx - 2] is not None:
                stores[idx - 2].wait()
            stage_ref[slot] = jnp.dot(
                chunk, bbf_ref[...], preferred_element_type=jnp.float32
            ).astype(jnp.bfloat16)
            cp = pltpu.make_async_copy(
                stage_ref.at[slot],
                out_ref.at[pl.ds(row0, half), :],
                store_sems.at[idx],
            )
            cp.start()
            stores[idx] = cp

        mm(0, a_ref[lo, :], my * m_per)
        mm(1, a_ref[hi, :], my * m_per + half)

        t1r_lo.wait_recv()
        t2r.start()
        mm(2, cl_ref[lo, :], left * m_per)

        t1l_hi.wait_recv()
        t2l.start()
        mm(3, cr_ref[hi, :], right * m_per + half)

        t1r_hi.wait_recv()
        mm(4, cl_ref[hi, :], left * m_per + half)

        t1l_lo.wait_recv()
        mm(5, cr_ref[lo, :], right * m_per)

        t2r.wait_recv()
        mm(6, co_ref[lo, :], opp * m_per)

        t2l.wait_recv()
        mm(7, co_ref[hi, :], opp * m_per + half)

        stores[6].wait()
        stores[7].wait()

        for t in (t1r_lo, t1r_hi, t1l_hi, t1l_lo, t2r, t2l):
            t.wait_send()

    return pl.pallas_call(
        body,
        out_shape=jax.ShapeDtypeStruct((N_DEV * m_per, n), jnp.bfloat16),
        in_specs=[
            pl.BlockSpec(memory_space=pltpu.VMEM),
            pl.BlockSpec(memory_space=pltpu.VMEM),
        ],
        out_specs=pl.BlockSpec(memory_space=pl.MemorySpace.ANY),
        scratch_shapes=[
            pltpu.VMEM((m_per, k), jnp.bfloat16),
            pltpu.VMEM((m_per, k), jnp.bfloat16),
            pltpu.VMEM((m_per, k), jnp.bfloat16),
            pltpu.VMEM((2, half, n), jnp.bfloat16),
            pltpu.VMEM((k, n), jnp.bfloat16),
            pltpu.VMEM((m_per, k), jnp.bfloat16),
            pltpu.SemaphoreType.DMA((6,)),
            pltpu.SemaphoreType.DMA((6,)),
            pltpu.SemaphoreType.DMA((8,)),
        ],
        compiler_params=pltpu.CompilerParams(
            collective_id=0,
            vmem_limit_bytes=100 * 1024 * 1024,
        ),
    )(A, B)


# device time: 110146 ns/iter; 1.1643x vs baseline; 1.0029x over previous
import jax
import jax.numpy as jnp
from jax import lax
from jax.experimental import pallas as pl
from jax.experimental.pallas import tpu as pltpu

N_DEV = 4


def kernel(A, B):
    m_per, k = A.shape
    _, n = B.shape
    half = m_per // 2

    def body(a32_ref, b32_ref, out_ref, cl_ref, cr_ref, co_ref,
             stage_ref, bbf_ref, a_ref, send_sems, recv_sems, store_sems):
        my = lax.axis_index("i")
        left = lax.rem(my + N_DEV - 1, N_DEV)
        right = lax.rem(my + 1, N_DEV)
        opp = lax.rem(my + 2, N_DEV)

        barrier_sem = pltpu.get_barrier_semaphore()
        for nbr in (left, right):
            pl.semaphore_signal(
                barrier_sem, inc=1,
                device_id=(nbr,), device_id_type=pl.DeviceIdType.MESH,
            )
        pl.semaphore_wait(barrier_sem, 2)

        def rdma(src, dst, sem_idx, target):
            return pltpu.make_async_remote_copy(
                src_ref=src, dst_ref=dst,
                send_sem=send_sems.at[sem_idx], recv_sem=recv_sems.at[sem_idx],
                device_id=(target,), device_id_type=pl.DeviceIdType.MESH,
            )

        quart = half // 2
        lo = pl.ds(0, half)
        hi = pl.ds(half, half)
        q = [pl.ds(i * quart, quart) for i in range(4)]
        t1r_lo = rdma(a_ref.at[lo], cl_ref.at[lo], 0, right)
        t1r_q2 = rdma(a_ref.at[q[2]], cl_ref.at[q[2]], 1, right)
        t1r_q3 = rdma(a_ref.at[q[3]], cl_ref.at[q[3]], 2, right)
        t1l_hi = rdma(a_ref.at[hi], cr_ref.at[hi], 3, left)
        t1l_q1 = rdma(a_ref.at[q[1]], cr_ref.at[q[1]], 4, left)
        t1l_q0 = rdma(a_ref.at[q[0]], cr_ref.at[q[0]], 5, left)
        t2r = rdma(cl_ref.at[lo], co_ref.at[lo], 6, right)
        t2l = rdma(cr_ref.at[hi], co_ref.at[hi], 7, left)

        a_ref[...] = a32_ref[...].astype(jnp.bfloat16)

        t1r_lo.start()
        t1l_hi.start()

        bbf_ref[...] = b32_ref[...].astype(jnp.bfloat16)

        stores = [None] * 10

        def mm(idx, chunk, row0, nrows):
            slot = idx % 2
            if stores[idx - 2] is not None:
                stores[idx - 2].wait()
            stage_ref[slot, 0:nrows, :] = jnp.dot(
                chunk, bbf_ref[...], preferred_element_type=jnp.float32
            ).astype(jnp.bfloat16)
            cp = pltpu.make_async_copy(
                stage_ref.at[slot, pl.ds(0, nrows), :],
                out_ref.at[pl.ds(row0, nrows), :],
                store_sems.at[idx],
            )
            cp.start()
            stores[idx] = cp

        mm(0, a_ref[lo, :], my * m_per, half)
        mm(1, a_ref[hi, :], my * m_per + half, half)

        t1r_lo.wait_recv()
        t2r.start()
        t1r_q2.start()
        t1r_q3.start()
        mm(2, cl_ref[lo, :], left * m_per, half)

        t1l_hi.wait_recv()
        t2l.start()
        t1l_q1.start()
        t1l_q0.start()
        mm(3, cr_ref[hi, :], right * m_per + half, half)

        t2r.wait_recv()
        mm(4, co_ref[lo, :], opp * m_per, half)

        t2l.wait_recv()
        mm(5, co_ref[hi, :], opp * m_per + half, half)

        t1r_q2.wait_recv()
        mm(6, cl_ref[q[2], :], left * m_per + 2 * quart, quart)

        t1l_q1.wait_recv()
        mm(7, cr_ref[q[1], :], right * m_per + quart, quart)

        t1r_q3.wait_recv()
        mm(8, cl_ref[q[3], :], left * m_per + 3 * quart, quart)

        t1l_q0.wait_recv()
        mm(9, cr_ref[q[0], :], right * m_per, quart)

        stores[8].wait()
        stores[9].wait()

        for t in (t1r_lo, t1r_q2, t1r_q3, t1l_hi, t1l_q1, t1l_q0, t2r, t2l):
            t.wait_send()

    return pl.pallas_call(
        body,
        out_shape=jax.ShapeDtypeStruct((N_DEV * m_per, n), jnp.bfloat16),
        in_specs=[
            pl.BlockSpec(memory_space=pltpu.VMEM),
            pl.BlockSpec(memory_space=pltpu.VMEM),
        ],
        out_specs=pl.BlockSpec(memory_space=pl.MemorySpace.ANY),
        scratch_shapes=[
            pltpu.VMEM((m_per, k), jnp.bfloat16),
            pltpu.VMEM((m_per, k), jnp.bfloat16),
            pltpu.VMEM((m_per, k), jnp.bfloat16),
            pltpu.VMEM((2, half, n), jnp.bfloat16),
            pltpu.VMEM((k, n), jnp.bfloat16),
            pltpu.VMEM((m_per, k), jnp.bfloat16),
            pltpu.SemaphoreType.DMA((8,)),
            pltpu.SemaphoreType.DMA((8,)),
            pltpu.SemaphoreType.DMA((10,)),
        ],
        compiler_params=pltpu.CompilerParams(
            collective_id=0,
            vmem_limit_bytes=100 * 1024 * 1024,
        ),
    )(A, B)
